# baseline (speedup 1.0000x reference)
import functools

import jax
import jax.numpy as jnp
from jax import lax
from jax.experimental import pallas as pl
from jax.experimental.pallas import tpu as pltpu
from jax.experimental.pallas import tpu_sc as plsc

N = 10000
NEMB = 128
SPLIT = 5120
NB = 5120
TAIL = N - SPLIT


_CHUNK = 80


def _make_sc_gather(b):
    info = plsc.get_sparse_core_info()
    nw = info.num_cores * info.num_subcores
    b_per_w = b // nw
    assert b_per_w % _CHUNK == 0 and b_per_w % 8 == 0
    n_chunks = b_per_w // _CHUNK
    mesh = plsc.VectorSubcoreMesh(core_axis_name="c", subcore_axis_name="s")

    @functools.partial(
        pl.kernel,
        mesh=mesh,
        out_type=jax.ShapeDtypeStruct((b, NEMB), jnp.float32),
        scratch_types=[
            pltpu.VMEM((b_per_w,), jnp.int32),
            pltpu.VMEM((b_per_w, NEMB), jnp.float32),
        ] + [pltpu.SemaphoreType.DMA] * (2 * n_chunks),
    )
    def gather_kernel(table_hbm, idx_hbm, out_hbm, idx_v, rows_v, *sems):
        gsems, osems = sems[:n_chunks], sems[n_chunks:]
        wid = lax.axis_index("s") * info.num_cores + lax.axis_index("c")
        base = wid * b_per_w
        pltpu.sync_copy(idx_hbm.at[pl.ds(base, b_per_w)], idx_v)
        gathers = [
            pltpu.async_copy(
                table_hbm.at[idx_v.at[pl.ds(j * _CHUNK, _CHUNK)]],
                rows_v.at[pl.ds(j * _CHUNK, _CHUNK)], gsems[j])
            for j in range(n_chunks)
        ]
        writes = []
        for j in range(n_chunks):
            gathers[j].wait()
            writes.append(pltpu.async_copy(
                rows_v.at[pl.ds(j * _CHUNK, _CHUNK)],
                out_hbm.at[pl.ds(base + j * _CHUNK, _CHUNK)], osems[j]))
        for wr in writes:
            wr.wait()

    return gather_kernel


_BM = 400


def _tc1_body(adj_ref, ea_ref, w1_ref, hp_ref, sa_s):
    m = pl.program_id(0)

    @pl.when(m == 0)
    def _init():
        sa_s[...] = jnp.dot(ea_ref[...], w1_ref[...],
                            preferred_element_type=jnp.float32)

    hp_ref[...] = jnp.dot(adj_ref[...], sa_s[...],
                          preferred_element_type=jnp.float32)


def _tc1(adj, e_a, w1):
    num_m = N // _BM
    return pl.pallas_call(
        _tc1_body,
        grid=(num_m,),
        in_specs=[
            pl.BlockSpec((_BM, SPLIT), lambda m: (m, 0)),
            pl.BlockSpec((NB, NEMB), lambda m: (0, 0)),
            pl.BlockSpec((NEMB, NEMB), lambda m: (0, 0)),
        ],
        out_specs=pl.BlockSpec((_BM, NEMB), lambda m: (m, 0)),
        out_shape=jax.ShapeDtypeStruct((N, NEMB), jnp.float32),
        scratch_shapes=[pltpu.VMEM((NB, NEMB), jnp.float32)],
        compiler_params=pltpu.CompilerParams(
            dimension_semantics=("arbitrary",)),
    )(adj, e_a, w1)


def _tc2_body(adj_ref, eb_ref, hp_ref, w1_ref, b1_ref, w2_ref, lw1_ref,
              lb1_ref, lw2_ref, lb2_ref, b2_ref, x_ref,
              sb_s, u_s, w_s, c_s):
    p = pl.program_id(0)
    m = pl.program_id(1)
    k = pl.program_id(2)

    @pl.when(jnp.logical_and(p == 0, jnp.logical_and(m == 0, k == 0)))
    def _init():
        sb = jnp.dot(eb_ref[...], w1_ref[...],
                     preferred_element_type=jnp.float32)
        rows = lax.broadcasted_iota(jnp.int32, (NB, NEMB), 0)
        sb_s[...] = jnp.where(rows < TAIL, sb, 0.0)
        t = jnp.dot(lw1_ref[...], lw2_ref[...],
                    preferred_element_type=jnp.float32)
        w_s[...] = jnp.dot(w2_ref[...], t,
                           preferred_element_type=jnp.float32)
        c_s[...] = (jnp.dot(jnp.dot(b2_ref[...], lw1_ref[...]), lw2_ref[...])
                    + jnp.dot(lb1_ref[...], lw2_ref[...]) + lb2_ref[...])
        u_s[...] = jnp.zeros_like(u_s)

    def masked_adj():
        cols = lax.broadcasted_iota(jnp.int32, (_BM, NB), 1)
        return jnp.where(cols < TAIL, adj_ref[...], 0.0)

    @pl.when(jnp.logical_and(p == 0, k == 1))
    def _phase0():
        h = (hp_ref[...]
             + jnp.dot(masked_adj(), sb_s[...],
                       preferred_element_type=jnp.float32)
             + b1_ref[...])
        r = jnp.maximum(h, 0.0)
        u = jnp.dot(r, w_s[...], preferred_element_type=jnp.float32)
        u_s[pl.ds(m * _BM, _BM), :] = u
        x_ref[...] = u

    @pl.when(jnp.logical_and(p == 1, k == 0))
    def _phase1a():
        x_ref[...] = jnp.dot(adj_ref[...], u_s[pl.ds(0, NB), :],
                             preferred_element_type=jnp.float32) + c_s[...]

    @pl.when(jnp.logical_and(p == 1, k == 1))
    def _phase1b():
        x_ref[...] += jnp.dot(masked_adj(), u_s[pl.ds(NB, NB), :],
                              preferred_element_type=jnp.float32)


def _tc2(adj, e_b, h_p, w1, b1, w2, lw1, lb1, lw2, lb2, b2):
    num_m = N // _BM
    grid = (2, num_m, 2)
    full = lambda shape: pl.BlockSpec(shape, lambda p, m, k: (0, 0))
    return pl.pallas_call(
        _tc2_body,
        grid=grid,
        in_specs=[
            pl.BlockSpec((_BM, NB),
                         lambda p, m, k: (m, jnp.where(p == 0, 1, k))),
            full((NB, NEMB)),
            pl.BlockSpec((_BM, NEMB),
                         lambda p, m, k: (jnp.where(p == 0, m, 0), 0)),
            full((NEMB, NEMB)),
            full((1, NEMB)),
            full((NEMB, NEMB)),
            full((NEMB, 16)),
            full((1, 16)),
            full((16, 1)),
            full((1, 1)),
            full((1, NEMB)),
        ],
        out_specs=pl.BlockSpec(
            (_BM, 1), lambda p, m, k: (jnp.where(p == 0, 0, m + 1), 0)),
        out_shape=jax.ShapeDtypeStruct((N + _BM, 1), jnp.float32),
        scratch_shapes=[
            pltpu.VMEM((NB, NEMB), jnp.float32),
            pltpu.VMEM((2 * NB, 1), jnp.float32),
            pltpu.VMEM((NEMB, 1), jnp.float32),
            pltpu.VMEM((1, 1), jnp.float32),
        ],
        compiler_params=pltpu.CompilerParams(
            dimension_semantics=("arbitrary", "arbitrary", "arbitrary")),
    )(adj, e_b, h_p, w1, b1, w2, lw1, lb1, lw2, lb2, b2)


def _sc_gather(emb_table, idx):
    return _make_sc_gather(idx.shape[0])(emb_table, idx)


def kernel(features, adj, emb_table, W1, b1, W2, b2, lw1, lb1, lw2, lb2):
    feats = features.astype(jnp.int32)
    idx_b = jnp.concatenate(
        [feats[SPLIT:], jnp.zeros((NB - TAIL,), jnp.int32)])
    e_a = _sc_gather(emb_table, feats[:SPLIT])
    e_b = _sc_gather(emb_table, idx_b)
    h_p = _tc1(adj, e_a, W1)
    x = _tc2(adj, e_b, h_p, W1, b1.reshape(1, -1), W2, lw1,
             lb1.reshape(1, -1), lw2, lb2.reshape(1, 1),
             b2.reshape(1, -1))[_BM:]
    user_emb = jnp.concatenate([e_a, e_b[:TAIL]], axis=0)
    return (x, user_emb)


# speedup vs baseline: 1.1120x; 1.1120x over previous
import functools

import jax
import jax.numpy as jnp
from jax import lax
from jax.experimental import pallas as pl
from jax.experimental.pallas import tpu as pltpu
from jax.experimental.pallas import tpu_sc as plsc

N = 10000
NEMB = 128


_CHUNKS = (128, 128, 64)


def _make_sc_gather(num_feat, b_pad):
    info = plsc.get_sparse_core_info()
    nw = info.num_cores * info.num_subcores
    b_per_w = b_pad // nw
    assert b_per_w == sum(_CHUNKS) and b_per_w % 8 == 0
    n_chunks = len(_CHUNKS)
    offs = [sum(_CHUNKS[:j]) for j in range(n_chunks)]
    mesh = plsc.VectorSubcoreMesh(core_axis_name="c", subcore_axis_name="s")

    @functools.partial(
        pl.kernel,
        mesh=mesh,
        out_type=jax.ShapeDtypeStruct((b_pad, NEMB), jnp.float32),
        scratch_types=[
            pltpu.VMEM((b_per_w,), jnp.int32),
            pltpu.VMEM((b_per_w, NEMB), jnp.float32),
        ] + [pltpu.SemaphoreType.DMA] * (n_chunks + 1),
    )
    def gather_kernel(table_hbm, idx_hbm, out_hbm, idx_v, rows_v, *sems):
        wid = lax.axis_index("s") * info.num_cores + lax.axis_index("c")
        base = wid * b_per_w
        pltpu.sync_copy(idx_hbm.at[pl.ds(base, b_per_w)], idx_v)
        gathers = [
            pltpu.async_copy(
                table_hbm.at[idx_v.at[pl.ds(offs[j], _CHUNKS[j])]],
                rows_v.at[pl.ds(offs[j], _CHUNKS[j])], sems[j])
            for j in range(n_chunks)
        ]
        for g in gathers:
            g.wait()
        pltpu.async_copy(rows_v, out_hbm.at[pl.ds(base, b_per_w)],
                         sems[n_chunks]).wait()

    return gather_kernel


_BM = 400


def _gcn_body(adj_ref, e_ref, w1_ref, b1_ref, w2_ref, lw1_ref, lb1_ref,
              lw2_ref, lb2_ref, b2_ref, x_ref, s_s, u_s, w_s, c_s):
    p = pl.program_id(0)
    m = pl.program_id(1)

    @pl.when(jnp.logical_and(p == 0, m == 0))
    def _init():
        s_s[...] = jnp.dot(e_ref[...], w1_ref[...],
                           preferred_element_type=jnp.float32)
        t = jnp.dot(lw1_ref[...], lw2_ref[...],
                    preferred_element_type=jnp.float32)
        w_s[...] = jnp.dot(w2_ref[...], t,
                           preferred_element_type=jnp.float32)
        c_s[...] = (jnp.dot(jnp.dot(b2_ref[...], lw1_ref[...]), lw2_ref[...])
                    + jnp.dot(lb1_ref[...], lw2_ref[...]) + lb2_ref[...])

    @pl.when(p == 0)
    def _phase0():
        h = jnp.dot(adj_ref[...], s_s[...],
                    preferred_element_type=jnp.float32) + b1_ref[...]
        r = jnp.maximum(h, 0.0)
        u = jnp.dot(r, w_s[...], preferred_element_type=jnp.float32)
        u_s[pl.ds(m * _BM, _BM), :] = u
        x_ref[...] = u

    @pl.when(p == 1)
    def _phase1():
        x_ref[...] = jnp.dot(adj_ref[...], u_s[...],
                             preferred_element_type=jnp.float32) + c_s[...]


def _gcn_pallas(adj, emb, w1, b1, w2, lw1, lb1, lw2, lb2, b2):
    n = adj.shape[0]
    num_m = n // _BM
    grid = (2, num_m)
    full = lambda shape: pl.BlockSpec(shape, lambda p, m: (0, 0))
    return pl.pallas_call(
        _gcn_body,
        grid=grid,
        in_specs=[
            pl.BlockSpec((_BM, n), lambda p, m: (m, 0)),
            full((n, NEMB)),
            full((NEMB, NEMB)),
            full((1, NEMB)),
            full((NEMB, NEMB)),
            full((NEMB, 16)),
            full((1, 16)),
            full((16, 1)),
            full((1, 1)),
            full((1, NEMB)),
        ],
        out_specs=pl.BlockSpec(
            (_BM, 1), lambda p, m: (jnp.where(p == 0, 0, m + 1), 0)),
        out_shape=jax.ShapeDtypeStruct((n + _BM, 1), jnp.float32),
        scratch_shapes=[
            pltpu.VMEM((n, NEMB), jnp.float32),
            pltpu.VMEM((n, 1), jnp.float32),
            pltpu.VMEM((NEMB, 1), jnp.float32),
            pltpu.VMEM((1, 1), jnp.float32),
        ],
        compiler_params=pltpu.CompilerParams(
            dimension_semantics=("arbitrary", "arbitrary")),
    )(adj, emb, w1, b1, w2, lw1, lb1, lw2, lb2, b2)


def _sc_gather(emb_table, idx_pad):
    return _make_sc_gather(emb_table.shape[0], idx_pad.shape[0])(
        emb_table, idx_pad)


def kernel(features, adj, emb_table, W1, b1, W2, b2, lw1, lb1, lw2, lb2):
    feats = features.astype(jnp.int32)
    b_pad = 10240
    idx_pad = jnp.concatenate(
        [feats, jnp.zeros((b_pad - N,), jnp.int32)])
    emb = _sc_gather(emb_table, idx_pad)
    user_emb = emb[:N]
    x = _gcn_pallas(adj, user_emb, W1, b1.reshape(1, -1), W2, lw1,
                    lb1.reshape(1, -1), lw2, lb2.reshape(1, 1),
                    b2.reshape(1, -1))[_BM:]
    return (x, user_emb)


# speedup vs baseline: 1.1791x; 1.0603x over previous
import functools

import jax
import jax.numpy as jnp
from jax import lax
from jax.experimental import pallas as pl
from jax.experimental.pallas import tpu as pltpu
from jax.experimental.pallas import tpu_sc as plsc

N = 10000
NEMB = 128


_CHUNKS = (128, 128, 64)
_BPW = sum(_CHUNKS)
_TAIL_ROWS = N - 31 * _BPW


def _make_sc_gather(num_feat):
    info = plsc.get_sparse_core_info()
    nw = info.num_cores * info.num_subcores
    assert nw * _BPW >= N and (nw - 1) * _BPW + _TAIL_ROWS == N
    n_chunks = len(_CHUNKS)
    offs = [sum(_CHUNKS[:j]) for j in range(n_chunks)]
    mesh = plsc.VectorSubcoreMesh(core_axis_name="c", subcore_axis_name="s")

    @functools.partial(
        pl.kernel,
        mesh=mesh,
        out_type=jax.ShapeDtypeStruct((N, NEMB), jnp.float32),
        scratch_types=[
            pltpu.VMEM((_BPW,), jnp.int32),
            pltpu.VMEM((_BPW, NEMB), jnp.float32),
        ] + [pltpu.SemaphoreType.DMA] * (n_chunks + 1),
    )
    def gather_kernel(table_hbm, idx_hbm, out_hbm, idx_v, rows_v, *sems):
        wid = lax.axis_index("s") * info.num_cores + lax.axis_index("c")
        base = wid * _BPW

        @pl.when(wid < nw - 1)
        def _full():
            pltpu.sync_copy(idx_hbm.at[pl.ds(base, _BPW)], idx_v)
            gathers = [
                pltpu.async_copy(
                    table_hbm.at[idx_v.at[pl.ds(offs[j], _CHUNKS[j])]],
                    rows_v.at[pl.ds(offs[j], _CHUNKS[j])], sems[j])
                for j in range(n_chunks)
            ]
            for g in gathers:
                g.wait()
            pltpu.async_copy(rows_v, out_hbm.at[pl.ds(base, _BPW)],
                             sems[n_chunks]).wait()

        @pl.when(wid == nw - 1)
        def _tail():
            pltpu.sync_copy(idx_hbm.at[pl.ds(base, _TAIL_ROWS)],
                            idx_v.at[pl.ds(0, _TAIL_ROWS)])
            pltpu.async_copy(
                table_hbm.at[idx_v.at[pl.ds(0, _TAIL_ROWS)]],
                rows_v.at[pl.ds(0, _TAIL_ROWS)], sems[0]).wait()
            pltpu.async_copy(rows_v.at[pl.ds(0, _TAIL_ROWS)],
                             out_hbm.at[pl.ds(base, _TAIL_ROWS)],
                             sems[n_chunks]).wait()

    return gather_kernel


_BM = 400


def _gcn_body(adj_ref, e_ref, w1_ref, b1_ref, w2_ref, lw1_ref, lb1_ref,
              lw2_ref, lb2_ref, b2_ref, x_ref, s_s, u_s, w_s, c_s):
    p = pl.program_id(0)
    m = pl.program_id(1)

    @pl.when(jnp.logical_and(p == 0, m == 0))
    def _init():
        s_s[...] = jnp.dot(e_ref[...], w1_ref[...],
                           preferred_element_type=jnp.float32)
        t = jnp.dot(lw1_ref[...], lw2_ref[...],
                    preferred_element_type=jnp.float32)
        w_s[...] = jnp.dot(w2_ref[...], t,
                           preferred_element_type=jnp.float32)
        c_s[...] = (jnp.dot(jnp.dot(b2_ref[...], lw1_ref[...]), lw2_ref[...])
                    + jnp.dot(lb1_ref[...], lw2_ref[...]) + lb2_ref[...])

    @pl.when(p == 0)
    def _phase0():
        h = jnp.dot(adj_ref[...], s_s[...],
                    preferred_element_type=jnp.float32) + b1_ref[...]
        r = jnp.maximum(h, 0.0)
        u = jnp.dot(r, w_s[...], preferred_element_type=jnp.float32)
        u_s[pl.ds(m * _BM, _BM), :] = u
        x_ref[...] = u

    @pl.when(p == 1)
    def _phase1():
        x_ref[...] = jnp.dot(adj_ref[...], u_s[...],
                             preferred_element_type=jnp.float32) + c_s[...]


def _gcn_pallas(adj, emb, w1, b1, w2, lw1, lb1, lw2, lb2, b2):
    n = adj.shape[0]
    num_m = n // _BM
    grid = (2, num_m)
    full = lambda shape: pl.BlockSpec(shape, lambda p, m: (0, 0))
    return pl.pallas_call(
        _gcn_body,
        grid=grid,
        in_specs=[
            pl.BlockSpec((_BM, n), lambda p, m: (m, 0)),
            full((n, NEMB)),
            full((NEMB, NEMB)),
            full((1, NEMB)),
            full((NEMB, NEMB)),
            full((NEMB, 16)),
            full((1, 16)),
            full((16, 1)),
            full((1, 1)),
            full((1, NEMB)),
        ],
        out_specs=pl.BlockSpec(
            (_BM, 1), lambda p, m: (jnp.where(p == 0, 0, m + 1), 0)),
        out_shape=jax.ShapeDtypeStruct((n + _BM, 1), jnp.float32),
        scratch_shapes=[
            pltpu.VMEM((n, NEMB), jnp.float32),
            pltpu.VMEM((n, 1), jnp.float32),
            pltpu.VMEM((NEMB, 1), jnp.float32),
            pltpu.VMEM((1, 1), jnp.float32),
        ],
        compiler_params=pltpu.CompilerParams(
            dimension_semantics=("arbitrary", "arbitrary")),
    )(adj, emb, w1, b1, w2, lw1, lb1, lw2, lb2, b2)


def _sc_gather(emb_table, idx):
    return _make_sc_gather(emb_table.shape[0])(emb_table, idx)


def kernel(features, adj, emb_table, W1, b1, W2, b2, lw1, lb1, lw2, lb2):
    feats = features.astype(jnp.int32)
    user_emb = _sc_gather(emb_table, feats)
    x = _gcn_pallas(adj, user_emb, W1, b1.reshape(1, -1), W2, lw1,
                    lb1.reshape(1, -1), lw2, lb2.reshape(1, 1),
                    b2.reshape(1, -1))[_BM:]
    return (x, user_emb)
